# baseline (speedup 1.0000x reference)
import functools

import jax
import jax.numpy as jnp
from jax import lax
from jax.experimental import pallas as pl
from jax.experimental.pallas import tpu as pltpu


def _ab_body(x_ref, wg_ref, af_ref, bf_ref, out_ref, *, A, E, R):
    x = x_ref[...]
    Bt = x.shape[0]
    ER = E * R
    logits = jnp.dot(x, wg_ref[...], preferred_element_type=jnp.float32)
    m = jnp.max(logits, axis=1, keepdims=True)
    iota_e = lax.broadcasted_iota(jnp.int32, (Bt, E), 1)
    e_idx = jnp.min(jnp.where(logits == m, iota_e, E), axis=1, keepdims=True)
    h = jnp.dot(x, af_ref[...], preferred_element_type=jnp.float32)
    col_e = (lax.broadcasted_iota(jnp.int32, (Bt, A * ER), 1) // R) % E
    g = jnp.where(col_e == e_idx, h, 0.0)
    for a in range(A):
        out = jnp.dot(g[:, a * ER:(a + 1) * ER], bf_ref[a],
                      preferred_element_type=jnp.float32)
        out_ref[a, :, :] = out


def kernel(x, w_gate, lora_a, lora_b):
    B, C = x.shape
    A, E, R, _ = lora_a.shape
    a_flat = lora_a.transpose(3, 0, 1, 2).reshape(C, A * E * R)
    b_flat = lora_b.transpose(0, 1, 3, 2).reshape(A, E * R, C)
    Bt = 1024
    return pl.pallas_call(
        functools.partial(_ab_body, A=A, E=E, R=R),
        grid=(B // Bt,),
        in_specs=[
            pl.BlockSpec((Bt, C), lambda i: (i, 0)),
            pl.BlockSpec((C, E), lambda i: (0, 0)),
            pl.BlockSpec((C, A * E * R), lambda i: (0, 0)),
            pl.BlockSpec((A, E * R, C), lambda i: (0, 0, 0)),
        ],
        out_specs=pl.BlockSpec((A, Bt, C), lambda i: (0, i, 0)),
        out_shape=jax.ShapeDtypeStruct((A, B, C), jnp.float32),
        compiler_params=pltpu.CompilerParams(
            dimension_semantics=("arbitrary",),
        ),
    )(x, w_gate, a_flat, b_flat)
